# Initial kernel scaffold; baseline (speedup 1.0000x reference)
import jax
import jax.numpy as jnp
from jax.experimental import pallas as pl


def kernel(token_ids, emb, W, b):
    raise NotImplementedError("write your pallas kernel here")


# speedup vs baseline: 9.9341x; 9.9341x over previous
import functools

import jax
import jax.numpy as jnp
from jax import lax
from jax.experimental import pallas as pl
from jax.experimental.pallas import tpu as pltpu
from jax.experimental.pallas import tpu_sc as plsc

_NUM_CORES = 2
_NUM_SUBCORES = 16
_NW = _NUM_CORES * _NUM_SUBCORES
_LANES = 16


def _make_pool_kernel(B, Lseq, D):
    rows_per_w = B // _NW
    CR = 2
    chunk_len = CR * Lseq
    n_chunks = rows_per_w // CR
    n_main = n_chunks - 2
    nsub = D // _LANES
    scale = 1.0 / Lseq
    mesh = plsc.VectorSubcoreMesh(
        core_axis_name="c", subcore_axis_name="s",
        num_cores=_NUM_CORES, num_subcores=_NUM_SUBCORES)

    @functools.partial(
        pl.kernel,
        out_type=jax.ShapeDtypeStruct((B, D), jnp.float32),
        mesh=mesh,
        scratch_types=[
            pltpu.VMEM((n_chunks, chunk_len), jnp.int32),
            pltpu.VMEM((2, chunk_len, D), jnp.float32),
            pltpu.VMEM((rows_per_w, D), jnp.float32),
            pltpu.SemaphoreType.DMA,
            pltpu.SemaphoreType.DMA,
        ],
    )
    def pool(tok_hbm, emb_hbm, out_hbm, idx_v, rows_v, pooled_v, sem0, sem1):
        sems = (sem0, sem1)
        wid = lax.axis_index("s") * _NUM_CORES + lax.axis_index("c")
        base_row = wid * rows_per_w
        pltpu.sync_copy(tok_hbm.at[wid], idx_v)

        def start(chunk, b):
            pltpu.async_copy(emb_hbm.at[idx_v.at[chunk]], rows_v.at[b], sems[b])

        def wait(chunk, b):
            pltpu.make_async_copy(
                emb_hbm.at[idx_v.at[chunk]], rows_v.at[b], sems[b]).wait()

        def accumulate(chunk, b):
            for r in range(CR):
                def body(j, accs, r=r):
                    base = r * Lseq + j
                    return tuple(
                        accs[c] + rows_v[b, base, pl.ds(c * _LANES, _LANES)]
                        for c in range(nsub))
                accs = lax.fori_loop(
                    0, Lseq, body,
                    tuple(jnp.zeros((_LANES,), jnp.float32)
                          for _ in range(nsub)))
                row = chunk * CR + r
                for c in range(nsub):
                    pooled_v[row, pl.ds(c * _LANES, _LANES)] = accs[c] * scale

        start(0, 0)
        start(1, 1)

        def pair(i, carry):
            for b in range(2):
                chunk = 2 * i + b
                wait(chunk, b)
                accumulate(chunk, b)
                start(chunk + 2, b)
            return carry
        lax.fori_loop(0, n_main // 2, pair, 0)

        for b in range(2):
            chunk = n_main + b
            wait(chunk, b)
            accumulate(chunk, b)

        pltpu.sync_copy(pooled_v, out_hbm.at[pl.ds(base_row, rows_per_w)])

    return pool


def _project(pooled, W, b):
    B, D = pooled.shape
    M = W.shape[0]
    BLK = 512

    def mm(x_ref, w_ref, b_ref, o_ref):
        o_ref[...] = lax.dot_general(
            x_ref[...], w_ref[...], (((1,), (1,)), ((), ())),
            preferred_element_type=jnp.float32) + b_ref[...]

    return pl.pallas_call(
        mm,
        grid=(B // BLK,),
        in_specs=[
            pl.BlockSpec((BLK, D), lambda i: (i, 0)),
            pl.BlockSpec((M, D), lambda i: (0, 0)),
            pl.BlockSpec((1, M), lambda i: (0, 0)),
        ],
        out_specs=pl.BlockSpec((BLK, M), lambda i: (i, 0)),
        out_shape=jax.ShapeDtypeStruct((B, M), jnp.float32),
    )(pooled, W, b.reshape(1, M))


def kernel(token_ids, emb, W, b):
    B, Lseq = token_ids.shape
    tok = token_ids.astype(jnp.int32).reshape(
        _NW, (B // _NW) // 2, 2 * Lseq)
    pooled = _make_pool_kernel(B, Lseq, emb.shape[1])(tok, emb)
    out = _project(pooled, W, b)
    return out[:, None, :]
